# baseline (speedup 1.0000x reference)
import functools

import jax
import jax.numpy as jnp
from jax import lax
from jax.experimental import pallas as pl

N_NODES = 10000
BLK = 1000
GRID = N_NODES // BLK


def _enc_stage1(x0_ref, x1_ref, w0_ref, b0_ref, w1_ref, b1_ref,
                z0_ref, z1_ref, m0_ref, m1_ref):
    i = pl.program_id(0)
    for x_ref, w_ref, b_ref, z_ref, m_ref in (
            (x0_ref, w0_ref, b0_ref, z0_ref, m0_ref),
            (x1_ref, w1_ref, b1_ref, z1_ref, m1_ref)):
        z = jnp.dot(x_ref[...], w_ref[...],
                    preferred_element_type=jnp.float32) + b_ref[...]
        z_ref[...] = z
        mom = jnp.concatenate(
            [jnp.sum(z, axis=0, keepdims=True),
             jnp.sum(z * z, axis=0, keepdims=True)], axis=0)

        @pl.when(i == 0)
        def _():
            m_ref[...] = mom

        @pl.when(i > 0)
        def _():
            m_ref[...] += mom


def _enc_stage2(z0_ref, z1_ref, w0_ref, b0_ref, w1_ref, b1_ref,
                y0_ref, y1_ref, m0_ref, m1_ref):
    i = pl.program_id(0)
    for z_ref, w_ref, b_ref, y_ref, m_ref in (
            (z0_ref, w0_ref, b0_ref, y0_ref, m0_ref),
            (z1_ref, w1_ref, b1_ref, y1_ref, m1_ref)):
        y = jnp.dot(z_ref[...], w_ref[...],
                    preferred_element_type=jnp.float32) + b_ref[...]
        y_ref[...] = y
        mom = jnp.concatenate(
            [jnp.sum(y, axis=0, keepdims=True),
             jnp.sum(y * y, axis=0, keepdims=True)], axis=0)

        @pl.when(i == 0)
        def _():
            m_ref[...] = mom

        @pl.when(i > 0)
        def _():
            m_ref[...] += mom


def _enc_stage3(y0_ref, y1_ref, a0_ref, a1_ref, c_ref, wp_ref, bp_ref,
                feats_ref, pooled_ref):
    feats = (jnp.dot(y0_ref[...], a0_ref[...], preferred_element_type=jnp.float32)
             + jnp.dot(y1_ref[...], a1_ref[...], preferred_element_type=jnp.float32)
             + c_ref[...])
    feats_ref[...] = feats
    pooled_ref[...] = jax.nn.relu(
        jnp.dot(feats, wp_ref[...], preferred_element_type=jnp.float32)
        + bp_ref[...])


def _sage_mid(h_ref, agg_ref, ws_ref, wn_ref, b_ref, wp_ref, bp_ref,
              out_ref, pooled_ref):
    out = jax.nn.relu(
        jnp.dot(h_ref[...], ws_ref[...], preferred_element_type=jnp.float32)
        + jnp.dot(agg_ref[...], wn_ref[...], preferred_element_type=jnp.float32)
        + b_ref[...])
    out_ref[...] = out
    pooled_ref[...] = jax.nn.relu(
        jnp.dot(out, wp_ref[...], preferred_element_type=jnp.float32)
        + bp_ref[...])


def _sage_last(h_ref, agg_ref, ws_ref, wn_ref, b_ref, out_ref):
    out_ref[...] = (
        jnp.dot(h_ref[...], ws_ref[...], preferred_element_type=jnp.float32)
        + jnp.dot(agg_ref[...], wn_ref[...], preferred_element_type=jnp.float32)
        + b_ref[...])


def _row_spec(d):
    return pl.BlockSpec((BLK, d), lambda i: (i, 0))


def _full_spec(shape):
    nd = len(shape)
    return pl.BlockSpec(shape, lambda i: (0,) * nd)


def _bn_affine(mom, g, be, eps=1e-5):
    mean = mom[0] / N_NODES
    var = mom[1] / N_NODES - mean * mean
    s = g / jnp.sqrt(var + eps)
    t = be - mean * s
    return s, t


def _segment_max(pooled, src, dst):
    msg = pooled[src]
    agg = jax.ops.segment_max(msg, dst, num_segments=N_NODES)
    return jnp.where(jnp.isfinite(agg), agg, 0.0)


def kernel(x0, x1, edge_index, params):
    enc = params['enc']
    gnn = params['gnn']
    src = edge_index[0]
    dst = edge_index[1]

    d0 = x0.shape[1]
    d1 = x1.shape[1]
    h500 = enc[0]['W1'].shape[1]
    lat = enc[0]['W2'].shape[1]
    dd = enc[0]['Wd'].shape[1]

    z0, z1, m0, m1 = pl.pallas_call(
        _enc_stage1,
        grid=(GRID,),
        in_specs=[_row_spec(d0), _row_spec(d1),
                  _full_spec((d0, h500)), _full_spec((1, h500)),
                  _full_spec((d1, h500)), _full_spec((1, h500))],
        out_specs=[_row_spec(h500), _row_spec(h500),
                   _full_spec((2, h500)), _full_spec((2, h500))],
        out_shape=[jax.ShapeDtypeStruct((N_NODES, h500), jnp.float32),
                   jax.ShapeDtypeStruct((N_NODES, h500), jnp.float32),
                   jax.ShapeDtypeStruct((2, h500), jnp.float32),
                   jax.ShapeDtypeStruct((2, h500), jnp.float32)],
    )(x0, x1,
      enc[0]['W1'], enc[0]['b1'].reshape(1, -1),
      enc[1]['W1'], enc[1]['b1'].reshape(1, -1))

    w2f, b2f = [], []
    for m, mom in ((0, m0), (1, m1)):
        s, t = _bn_affine(mom, enc[m]['g1'], enc[m]['be1'])
        w2f.append(s[:, None] * enc[m]['W2'])
        b2f.append((t @ enc[m]['W2'] + enc[m]['b2']).reshape(1, -1))

    y0, y1, n0, n1 = pl.pallas_call(
        _enc_stage2,
        grid=(GRID,),
        in_specs=[_row_spec(h500), _row_spec(h500),
                  _full_spec((h500, lat)), _full_spec((1, lat)),
                  _full_spec((h500, lat)), _full_spec((1, lat))],
        out_specs=[_row_spec(lat), _row_spec(lat),
                   _full_spec((2, lat)), _full_spec((2, lat))],
        out_shape=[jax.ShapeDtypeStruct((N_NODES, lat), jnp.float32),
                   jax.ShapeDtypeStruct((N_NODES, lat), jnp.float32),
                   jax.ShapeDtypeStruct((2, lat), jnp.float32),
                   jax.ShapeDtypeStruct((2, lat), jnp.float32)],
    )(z0, z1, w2f[0], b2f[0], w2f[1], b2f[1])

    af, cf = [], 0.0
    for m, mom in ((0, n0), (1, n1)):
        s, t = _bn_affine(mom, enc[m]['g2'], enc[m]['be2'])
        af.append(0.5 * (s[:, None] * enc[m]['Wd']))
        cf = cf + 0.5 * (t @ enc[m]['Wd'] + enc[m]['bd'])
    cf = cf.reshape(1, -1)

    feats, pooled1 = pl.pallas_call(
        _enc_stage3,
        grid=(GRID,),
        in_specs=[_row_spec(lat), _row_spec(lat),
                  _full_spec((lat, dd)), _full_spec((lat, dd)),
                  _full_spec((1, dd)),
                  _full_spec((dd, dd)), _full_spec((1, dd))],
        out_specs=[_row_spec(dd), _row_spec(dd)],
        out_shape=[jax.ShapeDtypeStruct((N_NODES, dd), jnp.float32),
                   jax.ShapeDtypeStruct((N_NODES, dd), jnp.float32)],
    )(y0, y1, af[0], af[1], cf,
      gnn[0]['Wp'], gnn[0]['bp'].reshape(1, -1))

    agg1 = _segment_max(pooled1, src, dst)

    dmid = gnn[0]['Ws'].shape[1]
    out1, pooled2 = pl.pallas_call(
        _sage_mid,
        grid=(GRID,),
        in_specs=[_row_spec(dd), _row_spec(dd),
                  _full_spec((dd, dmid)), _full_spec((dd, dmid)),
                  _full_spec((1, dmid)),
                  _full_spec((dmid, dmid)), _full_spec((1, dmid))],
        out_specs=[_row_spec(dmid), _row_spec(dmid)],
        out_shape=[jax.ShapeDtypeStruct((N_NODES, dmid), jnp.float32),
                   jax.ShapeDtypeStruct((N_NODES, dmid), jnp.float32)],
    )(feats, agg1, gnn[0]['Ws'], gnn[0]['Wn'], gnn[0]['b'].reshape(1, -1),
      gnn[1]['Wp'], gnn[1]['bp'].reshape(1, -1))

    agg2 = _segment_max(pooled2, src, dst)

    dout = gnn[1]['Ws'].shape[1]
    out2 = pl.pallas_call(
        _sage_last,
        grid=(GRID,),
        in_specs=[_row_spec(dmid), _row_spec(dmid),
                  _full_spec((dmid, dout)), _full_spec((dmid, dout)),
                  _full_spec((1, dout))],
        out_specs=_row_spec(dout),
        out_shape=jax.ShapeDtypeStruct((N_NODES, dout), jnp.float32),
    )(out1, agg2, gnn[1]['Ws'], gnn[1]['Wn'], gnn[1]['b'].reshape(1, -1))

    return out2


# speedup vs baseline: 1.1316x; 1.1316x over previous
import functools

import jax
import jax.numpy as jnp
from jax import lax
from jax.experimental import pallas as pl
from jax.experimental.pallas import tpu as pltpu
from jax.experimental.pallas import tpu_sc as plsc

N_NODES = 10000
BLK = 1000
GRID = N_NODES // BLK

NUM_CORES = 2
NUM_SUBCORES = 16
NUM_W = NUM_CORES * NUM_SUBCORES
ROWS_PER_W = 320
N_PAD = NUM_W * ROWS_PER_W
D_FEAT = 128
N_EDGES = 320000
CHUNK = 6400
N_CHUNKS = N_EDGES // CHUNK
GROW = 64
CAP = CHUNK + GROW + 16


def _enc_stage1(x0_ref, x1_ref, w0_ref, b0_ref, w1_ref, b1_ref,
                z0_ref, z1_ref, m0_ref, m1_ref):
    i = pl.program_id(0)
    for x_ref, w_ref, b_ref, z_ref, m_ref in (
            (x0_ref, w0_ref, b0_ref, z0_ref, m0_ref),
            (x1_ref, w1_ref, b1_ref, z1_ref, m1_ref)):
        z = jnp.dot(x_ref[...], w_ref[...],
                    preferred_element_type=jnp.float32) + b_ref[...]
        z_ref[...] = z
        mom = jnp.concatenate(
            [jnp.sum(z, axis=0, keepdims=True),
             jnp.sum(z * z, axis=0, keepdims=True)], axis=0)

        @pl.when(i == 0)
        def _():
            m_ref[...] = mom

        @pl.when(i > 0)
        def _():
            m_ref[...] += mom


def _enc_stage2(z0_ref, z1_ref, w0_ref, b0_ref, w1_ref, b1_ref,
                y0_ref, y1_ref, m0_ref, m1_ref):
    i = pl.program_id(0)
    for z_ref, w_ref, b_ref, y_ref, m_ref in (
            (z0_ref, w0_ref, b0_ref, y0_ref, m0_ref),
            (z1_ref, w1_ref, b1_ref, y1_ref, m1_ref)):
        y = jnp.dot(z_ref[...], w_ref[...],
                    preferred_element_type=jnp.float32) + b_ref[...]
        y_ref[...] = y
        mom = jnp.concatenate(
            [jnp.sum(y, axis=0, keepdims=True),
             jnp.sum(y * y, axis=0, keepdims=True)], axis=0)

        @pl.when(i == 0)
        def _():
            m_ref[...] = mom

        @pl.when(i > 0)
        def _():
            m_ref[...] += mom


def _enc_stage3(y0_ref, y1_ref, a0_ref, a1_ref, c_ref, wp_ref, bp_ref,
                feats_ref, pooled_ref):
    feats = (jnp.dot(y0_ref[...], a0_ref[...], preferred_element_type=jnp.float32)
             + jnp.dot(y1_ref[...], a1_ref[...], preferred_element_type=jnp.float32)
             + c_ref[...])
    feats_ref[...] = feats
    pooled_ref[...] = jax.nn.relu(
        jnp.dot(feats, wp_ref[...], preferred_element_type=jnp.float32)
        + bp_ref[...])


def _sage_mid(h_ref, agg_ref, ws_ref, wn_ref, b_ref, wp_ref, bp_ref,
              out_ref, pooled_ref):
    out = jax.nn.relu(
        jnp.dot(h_ref[...], ws_ref[...], preferred_element_type=jnp.float32)
        + jnp.dot(agg_ref[...], wn_ref[...], preferred_element_type=jnp.float32)
        + b_ref[...])
    out_ref[...] = out
    pooled_ref[...] = jax.nn.relu(
        jnp.dot(out, wp_ref[...], preferred_element_type=jnp.float32)
        + bp_ref[...])


def _sage_last(h_ref, agg_ref, ws_ref, wn_ref, b_ref, out_ref):
    out_ref[...] = (
        jnp.dot(h_ref[...], ws_ref[...], preferred_element_type=jnp.float32)
        + jnp.dot(agg_ref[...], wn_ref[...], preferred_element_type=jnp.float32)
        + b_ref[...])


def _row_spec(d):
    return pl.BlockSpec((BLK, d), lambda i: (i, 0))


def _full_spec(shape):
    nd = len(shape)
    return pl.BlockSpec(shape, lambda i: (0,) * nd)


def _bn_affine(mom, g, be, eps=1e-5):
    mean = mom[0] / N_NODES
    var = mom[1] / N_NODES - mean * mean
    s = g / jnp.sqrt(var + eps)
    t = be - mean * s
    return s, t


def _segmax_body(pooled_hbm, src_hbm, dst_hbm, out_hbm,
                 sstage, dstage, plist, slist,
                 acc, rows0, rows1, sem0, sem1):
    wid = lax.axis_index("c") * NUM_SUBCORES + lax.axis_index("s")
    lo = wid * ROWS_PER_W
    iota16 = lax.iota(jnp.int32, 16)

    def zero_row(j, carry):
        for kk in range(8):
            acc[j, pl.ds(kk * 16, 16)] = jnp.zeros((16,), jnp.float32)
        return carry

    lax.fori_loop(0, ROWS_PER_W + 1, zero_row, 0)

    def accum16(lv, buf, base):
        for j in range(16):
            r = lv[j]
            for kk in range(8):
                sl = pl.ds(kk * 16, 16)
                acc[r, sl] = jnp.maximum(acc[r, sl], buf[base + j, sl])

    junk = lo * 512 + ROWS_PER_W

    def chunk_body(c, carry):
        pltpu.sync_copy(src_hbm.at[pl.ds(c * CHUNK, CHUNK)], sstage)
        pltpu.sync_copy(dst_hbm.at[pl.ds(c * CHUNK, CHUNK)], dstage)

        def filt(v, cur):
            ld = dstage[pl.ds(v * 16, 16)] - lo
            s = sstage[pl.ds(v * 16, 16)]
            m = (ld >= 0) & (ld < ROWS_PER_W)
            p = jnp.where(m, 1, 0)
            for k in (1, 2, 4, 8):
                sh = jnp.where(iota16 >= k, iota16 - k, 0)
                p = p + jnp.where(iota16 >= k, p[sh], 0)
            packed = jnp.where(m, s * 512 + ld, junk)
            target = iota16 + 1
            lowv = jnp.zeros((16,), jnp.int32)
            for st in (8, 4, 2, 1):
                cand = lowv + st
                cv = p[cand - 1]
                lowv = jnp.where(cv < target, cand, lowv)
            plist[pl.ds(cur, 16)] = packed[lowv]
            return cur + p[15]

        cur = lax.fori_loop(0, CHUNK // 16, filt, jnp.int32(0))
        for k in range(GROW // 16):
            plist[pl.ds(cur + k * 16, 16)] = jnp.zeros((16,), jnp.int32) + junk
        nbg = (cur + GROW - 1) // GROW

        def build_slist(t, carry2):
            for g in range(GROW // 16):
                sl = pl.ds(t * GROW + g * 16, 16)
                slist[sl] = lax.shift_right_logical(plist[sl], 9)
            return carry2

        lax.fori_loop(0, nbg, build_slist, 0)

        def fire(t, buf, sem):
            return pltpu.async_copy(
                pooled_hbm.at[slist.at[pl.ds(t * GROW, GROW)]], buf, sem)

        def process(t, buf):
            for g in range(GROW // 16):
                lv = jnp.bitwise_and(plist[pl.ds(t * GROW + g * 16, 16)], 511)
                accum16(lv, buf, g * 16)

        @pl.when(nbg > 0)
        def _():
            fire(jnp.int32(0), rows0, sem0)

        def pair(pp, carry2):
            t0 = pp * 2
            t1 = t0 + 1

            @pl.when(t1 < nbg)
            def _():
                fire(t1, rows1, sem1)

            pltpu.make_async_copy(pooled_hbm.at[slist.at[pl.ds(0, GROW)]],
                                  rows0, sem0).wait()
            process(t0, rows0)

            @pl.when(t0 + 2 < nbg)
            def _():
                fire(t0 + 2, rows0, sem0)

            @pl.when(t1 < nbg)
            def _():
                pltpu.make_async_copy(
                    pooled_hbm.at[slist.at[pl.ds(0, GROW)]],
                    rows1, sem1).wait()
                process(t1, rows1)
            return carry2

        lax.fori_loop(0, (nbg + 1) // 2, pair, 0)
        return carry

    lax.fori_loop(0, N_CHUNKS, chunk_body, 0)
    pltpu.sync_copy(acc.at[pl.ds(0, ROWS_PER_W)],
                    out_hbm.at[pl.ds(lo, ROWS_PER_W)])


def _segment_max(pooled, src, dst):
    mesh = plsc.VectorSubcoreMesh(core_axis_name="c", subcore_axis_name="s")
    f = pl.kernel(
        _segmax_body,
        out_type=jax.ShapeDtypeStruct((N_PAD, D_FEAT), jnp.float32),
        mesh=mesh,
        scratch_types=[
            pltpu.VMEM((CHUNK,), jnp.int32),
            pltpu.VMEM((CHUNK,), jnp.int32),
            pltpu.VMEM((CAP,), jnp.int32),
            pltpu.VMEM((CAP,), jnp.int32),
            pltpu.VMEM((ROWS_PER_W + 1, D_FEAT), jnp.float32),
            pltpu.VMEM((GROW, D_FEAT), jnp.float32),
            pltpu.VMEM((GROW, D_FEAT), jnp.float32),
            pltpu.SemaphoreType.DMA,
            pltpu.SemaphoreType.DMA,
        ],
    )
    return f(pooled, src, dst)


def kernel(x0, x1, edge_index, params):
    enc = params['enc']
    gnn = params['gnn']
    src = edge_index[0]
    dst = edge_index[1]

    d0 = x0.shape[1]
    d1 = x1.shape[1]
    h500 = enc[0]['W1'].shape[1]
    lat = enc[0]['W2'].shape[1]
    dd = enc[0]['Wd'].shape[1]

    z0, z1, m0, m1 = pl.pallas_call(
        _enc_stage1,
        grid=(GRID,),
        in_specs=[_row_spec(d0), _row_spec(d1),
                  _full_spec((d0, h500)), _full_spec((1, h500)),
                  _full_spec((d1, h500)), _full_spec((1, h500))],
        out_specs=[_row_spec(h500), _row_spec(h500),
                   _full_spec((2, h500)), _full_spec((2, h500))],
        out_shape=[jax.ShapeDtypeStruct((N_NODES, h500), jnp.float32),
                   jax.ShapeDtypeStruct((N_NODES, h500), jnp.float32),
                   jax.ShapeDtypeStruct((2, h500), jnp.float32),
                   jax.ShapeDtypeStruct((2, h500), jnp.float32)],
    )(x0, x1,
      enc[0]['W1'], enc[0]['b1'].reshape(1, -1),
      enc[1]['W1'], enc[1]['b1'].reshape(1, -1))

    w2f, b2f = [], []
    for m, mom in ((0, m0), (1, m1)):
        s, t = _bn_affine(mom, enc[m]['g1'], enc[m]['be1'])
        w2f.append(s[:, None] * enc[m]['W2'])
        b2f.append((t @ enc[m]['W2'] + enc[m]['b2']).reshape(1, -1))

    y0, y1, n0, n1 = pl.pallas_call(
        _enc_stage2,
        grid=(GRID,),
        in_specs=[_row_spec(h500), _row_spec(h500),
                  _full_spec((h500, lat)), _full_spec((1, lat)),
                  _full_spec((h500, lat)), _full_spec((1, lat))],
        out_specs=[_row_spec(lat), _row_spec(lat),
                   _full_spec((2, lat)), _full_spec((2, lat))],
        out_shape=[jax.ShapeDtypeStruct((N_NODES, lat), jnp.float32),
                   jax.ShapeDtypeStruct((N_NODES, lat), jnp.float32),
                   jax.ShapeDtypeStruct((2, lat), jnp.float32),
                   jax.ShapeDtypeStruct((2, lat), jnp.float32)],
    )(z0, z1, w2f[0], b2f[0], w2f[1], b2f[1])

    af, cf = [], 0.0
    for m, mom in ((0, n0), (1, n1)):
        s, t = _bn_affine(mom, enc[m]['g2'], enc[m]['be2'])
        af.append(0.5 * (s[:, None] * enc[m]['Wd']))
        cf = cf + 0.5 * (t @ enc[m]['Wd'] + enc[m]['bd'])
    cf = cf.reshape(1, -1)

    feats, pooled1 = pl.pallas_call(
        _enc_stage3,
        grid=(GRID,),
        in_specs=[_row_spec(lat), _row_spec(lat),
                  _full_spec((lat, dd)), _full_spec((lat, dd)),
                  _full_spec((1, dd)),
                  _full_spec((dd, dd)), _full_spec((1, dd))],
        out_specs=[_row_spec(dd), _row_spec(dd)],
        out_shape=[jax.ShapeDtypeStruct((N_NODES, dd), jnp.float32),
                   jax.ShapeDtypeStruct((N_NODES, dd), jnp.float32)],
    )(y0, y1, af[0], af[1], cf,
      gnn[0]['Wp'], gnn[0]['bp'].reshape(1, -1))

    agg1 = _segment_max(pooled1, src, dst)

    dmid = gnn[0]['Ws'].shape[1]
    out1, pooled2 = pl.pallas_call(
        _sage_mid,
        grid=(GRID,),
        in_specs=[_row_spec(dd), _row_spec(dd),
                  _full_spec((dd, dmid)), _full_spec((dd, dmid)),
                  _full_spec((1, dmid)),
                  _full_spec((dmid, dmid)), _full_spec((1, dmid))],
        out_specs=[_row_spec(dmid), _row_spec(dmid)],
        out_shape=[jax.ShapeDtypeStruct((N_NODES, dmid), jnp.float32),
                   jax.ShapeDtypeStruct((N_NODES, dmid), jnp.float32)],
    )(feats, agg1, gnn[0]['Ws'], gnn[0]['Wn'], gnn[0]['b'].reshape(1, -1),
      gnn[1]['Wp'], gnn[1]['bp'].reshape(1, -1))

    agg2 = _segment_max(pooled2, src, dst)

    dout = gnn[1]['Ws'].shape[1]
    out2 = pl.pallas_call(
        _sage_last,
        grid=(GRID,),
        in_specs=[_row_spec(dmid), _row_spec(dmid),
                  _full_spec((dmid, dout)), _full_spec((dmid, dout)),
                  _full_spec((1, dout))],
        out_specs=_row_spec(dout),
        out_shape=jax.ShapeDtypeStruct((N_NODES, dout), jnp.float32),
    )(out1, agg2, gnn[1]['Ws'], gnn[1]['Wn'], gnn[1]['b'].reshape(1, -1))

    return out2


# speedup vs baseline: 1.4287x; 1.2625x over previous
import functools

import jax
import jax.numpy as jnp
from jax import lax
from jax.experimental import pallas as pl
from jax.experimental.pallas import tpu as pltpu
from jax.experimental.pallas import tpu_sc as plsc

N_NODES = 10000
BLK = 1000
GRID = N_NODES // BLK

NUM_CORES = 2
NUM_SUBCORES = 16
NUM_W = NUM_CORES * NUM_SUBCORES
ROWS_PER_W = 320
N_PAD = NUM_W * ROWS_PER_W
D_FEAT = 128
N_EDGES = 320000
CHUNK = 6400
N_CHUNKS = N_EDGES // CHUNK
GROW = 64
CAP = CHUNK + GROW + 16


def _enc_stage1(x0_ref, x1_ref, w0_ref, b0_ref, w1_ref, b1_ref,
                z0_ref, z1_ref, m0_ref, m1_ref):
    i = pl.program_id(0)
    for x_ref, w_ref, b_ref, z_ref, m_ref in (
            (x0_ref, w0_ref, b0_ref, z0_ref, m0_ref),
            (x1_ref, w1_ref, b1_ref, z1_ref, m1_ref)):
        z = jnp.dot(x_ref[...], w_ref[...],
                    preferred_element_type=jnp.float32) + b_ref[...]
        z_ref[...] = z
        mom = jnp.concatenate(
            [jnp.sum(z, axis=0, keepdims=True),
             jnp.sum(z * z, axis=0, keepdims=True)], axis=0)

        @pl.when(i == 0)
        def _():
            m_ref[...] = mom

        @pl.when(i > 0)
        def _():
            m_ref[...] += mom


def _enc_stage2(z0_ref, z1_ref, w0_ref, b0_ref, w1_ref, b1_ref,
                y0_ref, y1_ref, m0_ref, m1_ref):
    i = pl.program_id(0)
    for z_ref, w_ref, b_ref, y_ref, m_ref in (
            (z0_ref, w0_ref, b0_ref, y0_ref, m0_ref),
            (z1_ref, w1_ref, b1_ref, y1_ref, m1_ref)):
        y = jnp.dot(z_ref[...], w_ref[...],
                    preferred_element_type=jnp.float32) + b_ref[...]
        y_ref[...] = y
        mom = jnp.concatenate(
            [jnp.sum(y, axis=0, keepdims=True),
             jnp.sum(y * y, axis=0, keepdims=True)], axis=0)

        @pl.when(i == 0)
        def _():
            m_ref[...] = mom

        @pl.when(i > 0)
        def _():
            m_ref[...] += mom


def _enc_stage3(y0_ref, y1_ref, a0_ref, a1_ref, c_ref, wp_ref, bp_ref,
                feats_ref, pooled_ref):
    feats = (jnp.dot(y0_ref[...], a0_ref[...], preferred_element_type=jnp.float32)
             + jnp.dot(y1_ref[...], a1_ref[...], preferred_element_type=jnp.float32)
             + c_ref[...])
    feats_ref[...] = feats
    pooled_ref[...] = jax.nn.relu(
        jnp.dot(feats, wp_ref[...], preferred_element_type=jnp.float32)
        + bp_ref[...])


def _sage_mid(h_ref, agg_ref, ws_ref, wn_ref, b_ref, wp_ref, bp_ref,
              out_ref, pooled_ref):
    out = jax.nn.relu(
        jnp.dot(h_ref[...], ws_ref[...], preferred_element_type=jnp.float32)
        + jnp.dot(agg_ref[...], wn_ref[...], preferred_element_type=jnp.float32)
        + b_ref[...])
    out_ref[...] = out
    pooled_ref[...] = jax.nn.relu(
        jnp.dot(out, wp_ref[...], preferred_element_type=jnp.float32)
        + bp_ref[...])


def _sage_last(h_ref, agg_ref, ws_ref, wn_ref, b_ref, out_ref):
    out_ref[...] = (
        jnp.dot(h_ref[...], ws_ref[...], preferred_element_type=jnp.float32)
        + jnp.dot(agg_ref[...], wn_ref[...], preferred_element_type=jnp.float32)
        + b_ref[...])


def _row_spec(d):
    return pl.BlockSpec((BLK, d), lambda i: (i, 0))


def _full_spec(shape):
    nd = len(shape)
    return pl.BlockSpec(shape, lambda i: (0,) * nd)


def _bn_affine(mom, g, be, eps=1e-5):
    mean = mom[0] / N_NODES
    var = mom[1] / N_NODES - mean * mean
    s = g / jnp.sqrt(var + eps)
    t = be - mean * s
    return s, t


def _segmax_body(pooled_hbm, src_hbm, dst_hbm, out_hbm,
                 sstage, dstage, plist, slist,
                 acc, rows0, rows1, sem0, sem1):
    wid = lax.axis_index("c") * NUM_SUBCORES + lax.axis_index("s")
    lo = wid * ROWS_PER_W
    iota16 = lax.iota(jnp.int32, 16)

    def zero_row(j, carry):
        for kk in range(8):
            acc[j, pl.ds(kk * 16, 16)] = jnp.zeros((16,), jnp.float32)
        return carry

    lax.fori_loop(0, ROWS_PER_W + 1, zero_row, 0)

    def accum16(lv, buf, base):
        for j in range(16):
            r = lv[j]
            for kk in range(8):
                sl = pl.ds(kk * 16, 16)
                acc[r, sl] = jnp.maximum(acc[r, sl], buf[base + j, sl])

    junk = lo * 512 + ROWS_PER_W

    def chunk_body(c, carry):
        pltpu.sync_copy(src_hbm.at[pl.ds(c * CHUNK, CHUNK)], sstage)
        pltpu.sync_copy(dst_hbm.at[pl.ds(c * CHUNK, CHUNK)], dstage)

        target = iota16 + 1

        def compact(base):
            ld = dstage[pl.ds(base, 16)] - lo
            s = sstage[pl.ds(base, 16)]
            m = (ld >= 0) & (ld < ROWS_PER_W)
            p = jnp.where(m, 1, 0)
            for k in (1, 2, 4, 8):
                sh = jnp.where(iota16 >= k, iota16 - k, 0)
                p = p + jnp.where(iota16 >= k, p[sh], 0)
            packed = jnp.where(m, s * 512 + ld, junk)
            lowv = jnp.zeros((16,), jnp.int32)
            for st in (8, 4, 2, 1):
                cand = lowv + st
                cv = p[cand - 1]
                lowv = jnp.where(cv < target, cand, lowv)
            return packed[lowv], p[15]

        def filt(v, cur):
            outs = [compact(v * 64 + q * 16) for q in range(4)]
            for comp, cnt in outs:
                plist[pl.ds(cur, 16)] = comp
                cur = cur + cnt
            return cur

        cur = lax.fori_loop(0, CHUNK // 64, filt, jnp.int32(0))
        for k in range(GROW // 16):
            plist[pl.ds(cur + k * 16, 16)] = jnp.zeros((16,), jnp.int32) + junk
        nbg = (cur + GROW - 1) // GROW

        def build_slist(t, carry2):
            for g in range(GROW // 16):
                sl = pl.ds(t * GROW + g * 16, 16)
                slist[sl] = lax.shift_right_logical(plist[sl], 9)
            return carry2

        lax.fori_loop(0, nbg, build_slist, 0)

        def fire(t, buf, sem):
            return pltpu.async_copy(
                pooled_hbm.at[slist.at[pl.ds(t * GROW, GROW)]], buf, sem)

        def process(t, buf):
            for g in range(GROW // 16):
                lv = jnp.bitwise_and(plist[pl.ds(t * GROW + g * 16, 16)], 511)
                accum16(lv, buf, g * 16)

        @pl.when(nbg > 0)
        def _():
            fire(jnp.int32(0), rows0, sem0)

        def pair(pp, carry2):
            t0 = pp * 2
            t1 = t0 + 1

            @pl.when(t1 < nbg)
            def _():
                fire(t1, rows1, sem1)

            pltpu.make_async_copy(pooled_hbm.at[slist.at[pl.ds(0, GROW)]],
                                  rows0, sem0).wait()
            process(t0, rows0)

            @pl.when(t0 + 2 < nbg)
            def _():
                fire(t0 + 2, rows0, sem0)

            @pl.when(t1 < nbg)
            def _():
                pltpu.make_async_copy(
                    pooled_hbm.at[slist.at[pl.ds(0, GROW)]],
                    rows1, sem1).wait()
                process(t1, rows1)
            return carry2

        lax.fori_loop(0, (nbg + 1) // 2, pair, 0)
        return carry

    lax.fori_loop(0, N_CHUNKS, chunk_body, 0)
    pltpu.sync_copy(acc.at[pl.ds(0, ROWS_PER_W)],
                    out_hbm.at[pl.ds(lo, ROWS_PER_W)])


def _segment_max(pooled, src, dst):
    mesh = plsc.VectorSubcoreMesh(core_axis_name="c", subcore_axis_name="s")
    f = pl.kernel(
        _segmax_body,
        out_type=jax.ShapeDtypeStruct((N_PAD, D_FEAT), jnp.float32),
        mesh=mesh,
        scratch_types=[
            pltpu.VMEM((CHUNK,), jnp.int32),
            pltpu.VMEM((CHUNK,), jnp.int32),
            pltpu.VMEM((CAP,), jnp.int32),
            pltpu.VMEM((CAP,), jnp.int32),
            pltpu.VMEM((ROWS_PER_W + 1, D_FEAT), jnp.float32),
            pltpu.VMEM((GROW, D_FEAT), jnp.float32),
            pltpu.VMEM((GROW, D_FEAT), jnp.float32),
            pltpu.SemaphoreType.DMA,
            pltpu.SemaphoreType.DMA,
        ],
    )
    return f(pooled, src, dst)


def kernel(x0, x1, edge_index, params):
    enc = params['enc']
    gnn = params['gnn']
    src = edge_index[0]
    dst = edge_index[1]

    d0 = x0.shape[1]
    d1 = x1.shape[1]
    h500 = enc[0]['W1'].shape[1]
    lat = enc[0]['W2'].shape[1]
    dd = enc[0]['Wd'].shape[1]

    z0, z1, m0, m1 = pl.pallas_call(
        _enc_stage1,
        grid=(GRID,),
        in_specs=[_row_spec(d0), _row_spec(d1),
                  _full_spec((d0, h500)), _full_spec((1, h500)),
                  _full_spec((d1, h500)), _full_spec((1, h500))],
        out_specs=[_row_spec(h500), _row_spec(h500),
                   _full_spec((2, h500)), _full_spec((2, h500))],
        out_shape=[jax.ShapeDtypeStruct((N_NODES, h500), jnp.float32),
                   jax.ShapeDtypeStruct((N_NODES, h500), jnp.float32),
                   jax.ShapeDtypeStruct((2, h500), jnp.float32),
                   jax.ShapeDtypeStruct((2, h500), jnp.float32)],
    )(x0, x1,
      enc[0]['W1'], enc[0]['b1'].reshape(1, -1),
      enc[1]['W1'], enc[1]['b1'].reshape(1, -1))

    w2f, b2f = [], []
    for m, mom in ((0, m0), (1, m1)):
        s, t = _bn_affine(mom, enc[m]['g1'], enc[m]['be1'])
        w2f.append(s[:, None] * enc[m]['W2'])
        b2f.append((t @ enc[m]['W2'] + enc[m]['b2']).reshape(1, -1))

    y0, y1, n0, n1 = pl.pallas_call(
        _enc_stage2,
        grid=(GRID,),
        in_specs=[_row_spec(h500), _row_spec(h500),
                  _full_spec((h500, lat)), _full_spec((1, lat)),
                  _full_spec((h500, lat)), _full_spec((1, lat))],
        out_specs=[_row_spec(lat), _row_spec(lat),
                   _full_spec((2, lat)), _full_spec((2, lat))],
        out_shape=[jax.ShapeDtypeStruct((N_NODES, lat), jnp.float32),
                   jax.ShapeDtypeStruct((N_NODES, lat), jnp.float32),
                   jax.ShapeDtypeStruct((2, lat), jnp.float32),
                   jax.ShapeDtypeStruct((2, lat), jnp.float32)],
    )(z0, z1, w2f[0], b2f[0], w2f[1], b2f[1])

    af, cf = [], 0.0
    for m, mom in ((0, n0), (1, n1)):
        s, t = _bn_affine(mom, enc[m]['g2'], enc[m]['be2'])
        af.append(0.5 * (s[:, None] * enc[m]['Wd']))
        cf = cf + 0.5 * (t @ enc[m]['Wd'] + enc[m]['bd'])
    cf = cf.reshape(1, -1)

    feats, pooled1 = pl.pallas_call(
        _enc_stage3,
        grid=(GRID,),
        in_specs=[_row_spec(lat), _row_spec(lat),
                  _full_spec((lat, dd)), _full_spec((lat, dd)),
                  _full_spec((1, dd)),
                  _full_spec((dd, dd)), _full_spec((1, dd))],
        out_specs=[_row_spec(dd), _row_spec(dd)],
        out_shape=[jax.ShapeDtypeStruct((N_NODES, dd), jnp.float32),
                   jax.ShapeDtypeStruct((N_NODES, dd), jnp.float32)],
    )(y0, y1, af[0], af[1], cf,
      gnn[0]['Wp'], gnn[0]['bp'].reshape(1, -1))

    agg1 = _segment_max(pooled1, src, dst)

    dmid = gnn[0]['Ws'].shape[1]
    out1, pooled2 = pl.pallas_call(
        _sage_mid,
        grid=(GRID,),
        in_specs=[_row_spec(dd), _row_spec(dd),
                  _full_spec((dd, dmid)), _full_spec((dd, dmid)),
                  _full_spec((1, dmid)),
                  _full_spec((dmid, dmid)), _full_spec((1, dmid))],
        out_specs=[_row_spec(dmid), _row_spec(dmid)],
        out_shape=[jax.ShapeDtypeStruct((N_NODES, dmid), jnp.float32),
                   jax.ShapeDtypeStruct((N_NODES, dmid), jnp.float32)],
    )(feats, agg1, gnn[0]['Ws'], gnn[0]['Wn'], gnn[0]['b'].reshape(1, -1),
      gnn[1]['Wp'], gnn[1]['bp'].reshape(1, -1))

    agg2 = _segment_max(pooled2, src, dst)

    dout = gnn[1]['Ws'].shape[1]
    out2 = pl.pallas_call(
        _sage_last,
        grid=(GRID,),
        in_specs=[_row_spec(dmid), _row_spec(dmid),
                  _full_spec((dmid, dout)), _full_spec((dmid, dout)),
                  _full_spec((1, dout))],
        out_specs=_row_spec(dout),
        out_shape=jax.ShapeDtypeStruct((N_NODES, dout), jnp.float32),
    )(out1, agg2, gnn[1]['Ws'], gnn[1]['Wn'], gnn[1]['b'].reshape(1, -1))

    return out2


# speedup vs baseline: 1.7726x; 1.2408x over previous
import functools

import jax
import jax.numpy as jnp
from jax import lax
from jax.experimental import pallas as pl
from jax.experimental.pallas import tpu as pltpu
from jax.experimental.pallas import tpu_sc as plsc

N_NODES = 10000
BLK = 1000
GRID = N_NODES // BLK

NUM_CORES = 2
NUM_SUBCORES = 16
NUM_W = NUM_CORES * NUM_SUBCORES
ROWS_PER_W = 320
N_PAD = NUM_W * ROWS_PER_W
D_FEAT = 128
N_EDGES = 320000
CHUNK = 6400
N_CHUNKS = N_EDGES // CHUNK
GROW = 32
CAP = CHUNK + GROW + 16


def _enc_stage1(x0_ref, x1_ref, w0_ref, b0_ref, w1_ref, b1_ref,
                z0_ref, z1_ref, m0_ref, m1_ref):
    i = pl.program_id(0)
    for x_ref, w_ref, b_ref, z_ref, m_ref in (
            (x0_ref, w0_ref, b0_ref, z0_ref, m0_ref),
            (x1_ref, w1_ref, b1_ref, z1_ref, m1_ref)):
        z = jnp.dot(x_ref[...], w_ref[...],
                    preferred_element_type=jnp.float32) + b_ref[...]
        z_ref[...] = z
        mom = jnp.concatenate(
            [jnp.sum(z, axis=0, keepdims=True),
             jnp.sum(z * z, axis=0, keepdims=True)], axis=0)

        @pl.when(i == 0)
        def _():
            m_ref[...] = mom

        @pl.when(i > 0)
        def _():
            m_ref[...] += mom


def _enc_stage2(z0_ref, z1_ref, w0_ref, b0_ref, w1_ref, b1_ref,
                y0_ref, y1_ref, m0_ref, m1_ref):
    i = pl.program_id(0)
    for z_ref, w_ref, b_ref, y_ref, m_ref in (
            (z0_ref, w0_ref, b0_ref, y0_ref, m0_ref),
            (z1_ref, w1_ref, b1_ref, y1_ref, m1_ref)):
        y = jnp.dot(z_ref[...], w_ref[...],
                    preferred_element_type=jnp.float32) + b_ref[...]
        y_ref[...] = y
        mom = jnp.concatenate(
            [jnp.sum(y, axis=0, keepdims=True),
             jnp.sum(y * y, axis=0, keepdims=True)], axis=0)

        @pl.when(i == 0)
        def _():
            m_ref[...] = mom

        @pl.when(i > 0)
        def _():
            m_ref[...] += mom


def _enc_stage3(y0_ref, y1_ref, a0_ref, a1_ref, c_ref, wp_ref, bp_ref,
                feats_ref, pooled_ref):
    feats = (jnp.dot(y0_ref[...], a0_ref[...], preferred_element_type=jnp.float32)
             + jnp.dot(y1_ref[...], a1_ref[...], preferred_element_type=jnp.float32)
             + c_ref[...])
    feats_ref[...] = feats
    pooled_ref[...] = jax.nn.relu(
        jnp.dot(feats, wp_ref[...], preferred_element_type=jnp.float32)
        + bp_ref[...])


def _sage_mid(h_ref, agg_ref, ws_ref, wn_ref, b_ref, wp_ref, bp_ref,
              out_ref, pooled_ref):
    out = jax.nn.relu(
        jnp.dot(h_ref[...], ws_ref[...], preferred_element_type=jnp.float32)
        + jnp.dot(agg_ref[...], wn_ref[...], preferred_element_type=jnp.float32)
        + b_ref[...])
    out_ref[...] = out
    pooled_ref[...] = jax.nn.relu(
        jnp.dot(out, wp_ref[...], preferred_element_type=jnp.float32)
        + bp_ref[...])


def _sage_last(h_ref, agg_ref, ws_ref, wn_ref, b_ref, out_ref):
    out_ref[...] = (
        jnp.dot(h_ref[...], ws_ref[...], preferred_element_type=jnp.float32)
        + jnp.dot(agg_ref[...], wn_ref[...], preferred_element_type=jnp.float32)
        + b_ref[...])


def _row_spec(d):
    return pl.BlockSpec((BLK, d), lambda i: (i, 0))


def _full_spec(shape):
    nd = len(shape)
    return pl.BlockSpec(shape, lambda i: (0,) * nd)


def _bn_affine(mom, g, be, eps=1e-5):
    mean = mom[0] / N_NODES
    var = mom[1] / N_NODES - mean * mean
    s = g / jnp.sqrt(var + eps)
    t = be - mean * s
    return s, t


def _segmax_body(pooled_hbm, src_hbm, dst_hbm, out_hbm,
                 sstage, dstage, plist, slist,
                 acc, acc2, rows0, rows1, sem0, sem1):
    wid = lax.axis_index("c") * NUM_SUBCORES + lax.axis_index("s")
    lo = wid * ROWS_PER_W
    iota16 = lax.iota(jnp.int32, 16)

    def zero_row(j, carry):
        for kk in range(8):
            acc[j, pl.ds(kk * 16, 16)] = jnp.zeros((16,), jnp.float32)
            acc2[j, pl.ds(kk * 16, 16)] = jnp.zeros((16,), jnp.float32)
        return carry

    lax.fori_loop(0, ROWS_PER_W + 1, zero_row, 0)

    def accum16(lv, buf, base):
        for j in range(16):
            bank = acc if j % 2 == 0 else acc2
            r = lv[j]
            for kk in range(8):
                sl = pl.ds(kk * 16, 16)
                bank[r, sl] = jnp.maximum(bank[r, sl], buf[base + j, sl])

    junk = lo * 512 + ROWS_PER_W

    def chunk_body(c, carry):
        pltpu.sync_copy(src_hbm.at[pl.ds(c * CHUNK, CHUNK)], sstage)
        pltpu.sync_copy(dst_hbm.at[pl.ds(c * CHUNK, CHUNK)], dstage)

        target = iota16 + 1

        def compact(base):
            ld = dstage[pl.ds(base, 16)] - lo
            s = sstage[pl.ds(base, 16)]
            m = (ld >= 0) & (ld < ROWS_PER_W)
            p = jnp.where(m, 1, 0)
            for k in (1, 2, 4, 8):
                sh = jnp.where(iota16 >= k, iota16 - k, 0)
                p = p + jnp.where(iota16 >= k, p[sh], 0)
            packed = jnp.where(m, s * 512 + ld, junk)
            lowv = jnp.zeros((16,), jnp.int32)
            for st in (8, 4, 2, 1):
                cand = lowv + st
                cv = p[cand - 1]
                lowv = jnp.where(cv < target, cand, lowv)
            return packed[lowv], p[15]

        def filt(v, cur):
            outs = [compact(v * 64 + q * 16) for q in range(4)]
            for comp, cnt in outs:
                plist[pl.ds(cur, 16)] = comp
                cur = cur + cnt
            return cur

        cur = lax.fori_loop(0, CHUNK // 64, filt, jnp.int32(0))
        for k in range(GROW // 16):
            plist[pl.ds(cur + k * 16, 16)] = jnp.zeros((16,), jnp.int32) + junk
        nbg = (cur + GROW - 1) // GROW

        def build_slist(t, carry2):
            for g in range(GROW // 16):
                sl = pl.ds(t * GROW + g * 16, 16)
                slist[sl] = lax.shift_right_logical(plist[sl], 9)
            return carry2

        lax.fori_loop(0, nbg, build_slist, 0)

        def fire(t, buf, sem):
            return pltpu.async_copy(
                pooled_hbm.at[slist.at[pl.ds(t * GROW, GROW)]], buf, sem)

        def process(t, buf):
            for g in range(GROW // 16):
                lv = jnp.bitwise_and(plist[pl.ds(t * GROW + g * 16, 16)], 511)
                accum16(lv, buf, g * 16)

        @pl.when(nbg > 0)
        def _():
            fire(jnp.int32(0), rows0, sem0)

        def pair(pp, carry2):
            t0 = pp * 2
            t1 = t0 + 1

            @pl.when(t1 < nbg)
            def _():
                fire(t1, rows1, sem1)

            pltpu.make_async_copy(pooled_hbm.at[slist.at[pl.ds(0, GROW)]],
                                  rows0, sem0).wait()
            process(t0, rows0)

            @pl.when(t0 + 2 < nbg)
            def _():
                fire(t0 + 2, rows0, sem0)

            @pl.when(t1 < nbg)
            def _():
                pltpu.make_async_copy(
                    pooled_hbm.at[slist.at[pl.ds(0, GROW)]],
                    rows1, sem1).wait()
                process(t1, rows1)
            return carry2

        lax.fori_loop(0, (nbg + 1) // 2, pair, 0)
        return carry

    lax.fori_loop(0, N_CHUNKS, chunk_body, 0)

    def merge_row(j, carry):
        for kk in range(8):
            sl = pl.ds(kk * 16, 16)
            acc[j, sl] = jnp.maximum(acc[j, sl], acc2[j, sl])
        return carry

    lax.fori_loop(0, ROWS_PER_W, merge_row, 0)
    pltpu.sync_copy(acc.at[pl.ds(0, ROWS_PER_W)],
                    out_hbm.at[pl.ds(lo, ROWS_PER_W)])


def _segment_max(pooled, src, dst):
    mesh = plsc.VectorSubcoreMesh(core_axis_name="c", subcore_axis_name="s")
    f = pl.kernel(
        _segmax_body,
        out_type=jax.ShapeDtypeStruct((N_PAD, D_FEAT), jnp.float32),
        mesh=mesh,
        scratch_types=[
            pltpu.VMEM((CHUNK,), jnp.int32),
            pltpu.VMEM((CHUNK,), jnp.int32),
            pltpu.VMEM((CAP,), jnp.int32),
            pltpu.VMEM((CAP,), jnp.int32),
            pltpu.VMEM((ROWS_PER_W + 1, D_FEAT), jnp.float32),
            pltpu.VMEM((ROWS_PER_W + 1, D_FEAT), jnp.float32),
            pltpu.VMEM((GROW, D_FEAT), jnp.float32),
            pltpu.VMEM((GROW, D_FEAT), jnp.float32),
            pltpu.SemaphoreType.DMA,
            pltpu.SemaphoreType.DMA,
        ],
    )
    return f(pooled, src, dst)


def kernel(x0, x1, edge_index, params):
    enc = params['enc']
    gnn = params['gnn']
    src = edge_index[0]
    dst = edge_index[1]

    d0 = x0.shape[1]
    d1 = x1.shape[1]
    h500 = enc[0]['W1'].shape[1]
    lat = enc[0]['W2'].shape[1]
    dd = enc[0]['Wd'].shape[1]

    z0, z1, m0, m1 = pl.pallas_call(
        _enc_stage1,
        grid=(GRID,),
        in_specs=[_row_spec(d0), _row_spec(d1),
                  _full_spec((d0, h500)), _full_spec((1, h500)),
                  _full_spec((d1, h500)), _full_spec((1, h500))],
        out_specs=[_row_spec(h500), _row_spec(h500),
                   _full_spec((2, h500)), _full_spec((2, h500))],
        out_shape=[jax.ShapeDtypeStruct((N_NODES, h500), jnp.float32),
                   jax.ShapeDtypeStruct((N_NODES, h500), jnp.float32),
                   jax.ShapeDtypeStruct((2, h500), jnp.float32),
                   jax.ShapeDtypeStruct((2, h500), jnp.float32)],
    )(x0, x1,
      enc[0]['W1'], enc[0]['b1'].reshape(1, -1),
      enc[1]['W1'], enc[1]['b1'].reshape(1, -1))

    w2f, b2f = [], []
    for m, mom in ((0, m0), (1, m1)):
        s, t = _bn_affine(mom, enc[m]['g1'], enc[m]['be1'])
        w2f.append(s[:, None] * enc[m]['W2'])
        b2f.append((t @ enc[m]['W2'] + enc[m]['b2']).reshape(1, -1))

    y0, y1, n0, n1 = pl.pallas_call(
        _enc_stage2,
        grid=(GRID,),
        in_specs=[_row_spec(h500), _row_spec(h500),
                  _full_spec((h500, lat)), _full_spec((1, lat)),
                  _full_spec((h500, lat)), _full_spec((1, lat))],
        out_specs=[_row_spec(lat), _row_spec(lat),
                   _full_spec((2, lat)), _full_spec((2, lat))],
        out_shape=[jax.ShapeDtypeStruct((N_NODES, lat), jnp.float32),
                   jax.ShapeDtypeStruct((N_NODES, lat), jnp.float32),
                   jax.ShapeDtypeStruct((2, lat), jnp.float32),
                   jax.ShapeDtypeStruct((2, lat), jnp.float32)],
    )(z0, z1, w2f[0], b2f[0], w2f[1], b2f[1])

    af, cf = [], 0.0
    for m, mom in ((0, n0), (1, n1)):
        s, t = _bn_affine(mom, enc[m]['g2'], enc[m]['be2'])
        af.append(0.5 * (s[:, None] * enc[m]['Wd']))
        cf = cf + 0.5 * (t @ enc[m]['Wd'] + enc[m]['bd'])
    cf = cf.reshape(1, -1)

    feats, pooled1 = pl.pallas_call(
        _enc_stage3,
        grid=(GRID,),
        in_specs=[_row_spec(lat), _row_spec(lat),
                  _full_spec((lat, dd)), _full_spec((lat, dd)),
                  _full_spec((1, dd)),
                  _full_spec((dd, dd)), _full_spec((1, dd))],
        out_specs=[_row_spec(dd), _row_spec(dd)],
        out_shape=[jax.ShapeDtypeStruct((N_NODES, dd), jnp.float32),
                   jax.ShapeDtypeStruct((N_NODES, dd), jnp.float32)],
    )(y0, y1, af[0], af[1], cf,
      gnn[0]['Wp'], gnn[0]['bp'].reshape(1, -1))

    agg1 = _segment_max(pooled1, src, dst)

    dmid = gnn[0]['Ws'].shape[1]
    out1, pooled2 = pl.pallas_call(
        _sage_mid,
        grid=(GRID,),
        in_specs=[_row_spec(dd), _row_spec(dd),
                  _full_spec((dd, dmid)), _full_spec((dd, dmid)),
                  _full_spec((1, dmid)),
                  _full_spec((dmid, dmid)), _full_spec((1, dmid))],
        out_specs=[_row_spec(dmid), _row_spec(dmid)],
        out_shape=[jax.ShapeDtypeStruct((N_NODES, dmid), jnp.float32),
                   jax.ShapeDtypeStruct((N_NODES, dmid), jnp.float32)],
    )(feats, agg1, gnn[0]['Ws'], gnn[0]['Wn'], gnn[0]['b'].reshape(1, -1),
      gnn[1]['Wp'], gnn[1]['bp'].reshape(1, -1))

    agg2 = _segment_max(pooled2, src, dst)

    dout = gnn[1]['Ws'].shape[1]
    out2 = pl.pallas_call(
        _sage_last,
        grid=(GRID,),
        in_specs=[_row_spec(dmid), _row_spec(dmid),
                  _full_spec((dmid, dout)), _full_spec((dmid, dout)),
                  _full_spec((1, dout))],
        out_specs=_row_spec(dout),
        out_shape=jax.ShapeDtypeStruct((N_NODES, dout), jnp.float32),
    )(out1, agg2, gnn[1]['Ws'], gnn[1]['Wn'], gnn[1]['b'].reshape(1, -1))

    return out2
